# Initial kernel scaffold; baseline (speedup 1.0000x reference)
import jax
import jax.numpy as jnp
from jax.experimental import pallas as pl


def kernel(token_ids, weight):
    raise NotImplementedError("write your pallas kernel here")


# speedup vs baseline: 3.3501x; 3.3501x over previous
import functools

import jax
import jax.numpy as jnp
from jax import lax
from jax.experimental import pallas as pl
from jax.experimental.pallas import tpu as pltpu
from jax.experimental.pallas import tpu_sc as plsc

_NUM_CORES = 2
_NUM_SUBCORES = 16
_NUM_WORKERS = _NUM_CORES * _NUM_SUBCORES


@functools.lru_cache(maxsize=None)
def _make_gather_kernel(num_rows: int, dim: int, chunk: int):
    rows_per_worker = num_rows // _NUM_WORKERS
    num_chunks = rows_per_worker // chunk
    assert num_rows % _NUM_WORKERS == 0
    assert rows_per_worker % chunk == 0
    assert num_chunks % 2 == 0 and num_chunks >= 4
    assert chunk % 8 == 0

    mesh = plsc.VectorSubcoreMesh(
        core_axis_name="c",
        subcore_axis_name="s",
        num_cores=_NUM_CORES,
        num_subcores=_NUM_SUBCORES,
    )

    @functools.partial(
        pl.kernel,
        mesh=mesh,
        out_type=jax.ShapeDtypeStruct((num_rows, dim), jnp.float32),
        scratch_types=[
            pltpu.VMEM((rows_per_worker,), jnp.int32),
            pltpu.VMEM((chunk, dim), jnp.float32),
            pltpu.VMEM((chunk, dim), jnp.float32),
            pltpu.SemaphoreType.DMA,
            pltpu.SemaphoreType.DMA,
        ],
    )
    def gather_kernel(table_hbm, idx_hbm, out_hbm, idx_v, buf0, buf1, g0, g1):
        wid = lax.axis_index("s") * _NUM_CORES + lax.axis_index("c")
        base = wid * rows_per_worker
        pltpu.sync_copy(idx_hbm.at[pl.ds(base, rows_per_worker)], idx_v)

        def start_gather(chunk_id, buf, sem):
            off = chunk_id * chunk
            pltpu.async_copy(table_hbm.at[idx_v.at[pl.ds(off, chunk)]], buf, sem)

        def wait_gather(buf, sem):
            pltpu.make_async_copy(table_hbm.at[pl.ds(0, chunk)], buf, sem).wait()

        def store(chunk_id, buf):
            pltpu.sync_copy(buf, out_hbm.at[pl.ds(base + chunk_id * chunk, chunk)])

        start_gather(0, buf0, g0)
        start_gather(1, buf1, g1)

        def body(j, carry):
            i0 = 2 * j
            wait_gather(buf0, g0)
            store(i0, buf0)
            start_gather(i0 + 2, buf0, g0)
            wait_gather(buf1, g1)
            store(i0 + 1, buf1)
            start_gather(i0 + 3, buf1, g1)
            return carry

        lax.fori_loop(0, num_chunks // 2 - 1, body, 0, unroll=False)

        last = num_chunks - 2
        wait_gather(buf0, g0)
        store(last, buf0)
        wait_gather(buf1, g1)
        store(last + 1, buf1)

    return gather_kernel


def kernel(token_ids, weight):
    dim = weight.shape[1]
    idx = token_ids.reshape(-1).astype(jnp.int32)
    gather = _make_gather_kernel(idx.shape[0], dim, 400)
    out = gather(weight, idx)
    return out.reshape(token_ids.shape + (dim,))
